# Initial kernel scaffold; baseline (speedup 1.0000x reference)
import jax
import jax.numpy as jnp
from jax.experimental import pallas as pl


def kernel(logits, targets, epoch):
    raise NotImplementedError("write your pallas kernel here")


# speedup vs baseline: 28.5773x; 28.5773x over previous
import jax
import jax.numpy as jnp
from jax.experimental import pallas as pl
from jax.experimental.pallas import tpu as pltpu


def _body(epoch_ref, logits_ref, targets_ref, out_ref, bits_ref):
    l = logits_ref[...]
    t = targets_ref[...]
    nan_mask = jnp.isnan(t)
    t0 = jnp.where(nan_mask, 0.0, t)
    loss = jnp.maximum(l, 0.0) - l * t0 + jnp.log1p(jnp.exp(-jnp.abs(l)))
    ull = jnp.where(nan_mask, loss, 0.0)
    bits = jax.lax.bitcast_convert_type(ull, jnp.int32)
    bits_ref[...] = bits

    total = jnp.sum(loss)
    n = jnp.sum(jnp.where(nan_mask & (loss != 0.0), 1, 0).astype(jnp.int32))

    epoch = epoch_ref[0]
    percent = epoch.astype(jnp.float32) * 0.1 / 100.0
    num = n * epoch
    q = num // 1000
    r = num - q * 1000
    k = jnp.where(r > 500, q + 1, jnp.where(r == 500, q + (q % 2), q))
    k = jnp.where(percent > 1.0, n, k)
    k = jnp.where(k == 0, 1, k)

    def step(_, carry):
        lo, hi = carry
        mid = lo + ((hi - lo) >> 1)
        c = jnp.sum(jnp.where(bits_ref[...] >= mid, 1, 0).astype(jnp.int32))
        ge = c >= k
        return jnp.where(ge, mid, lo), jnp.where(ge, hi, mid)

    lo, _ = jax.lax.fori_loop(
        0, 31, step, (jnp.int32(0), jnp.int32(0x7F800000))
    )

    b2 = bits_ref[...]
    dropped = jnp.sum(
        jnp.where(b2 > lo, jax.lax.bitcast_convert_type(b2, jnp.float32), 0.0)
    )
    out_ref[0] = (total - dropped) / float(l.size)


def kernel(logits, targets, epoch):
    epoch_arr = jnp.asarray(epoch, jnp.int32).reshape((1,))
    out = pl.pallas_call(
        _body,
        out_shape=jax.ShapeDtypeStruct((1,), jnp.float32),
        in_specs=[
            pl.BlockSpec(memory_space=pltpu.SMEM),
            pl.BlockSpec(memory_space=pltpu.VMEM),
            pl.BlockSpec(memory_space=pltpu.VMEM),
        ],
        out_specs=pl.BlockSpec(memory_space=pltpu.SMEM),
        scratch_shapes=[pltpu.VMEM(logits.shape, jnp.int32)],
    )(epoch_arr, logits, targets)
    return out[0]
